# Initial kernel scaffold; baseline (speedup 1.0000x reference)
import jax
import jax.numpy as jnp
from jax.experimental import pallas as pl


def kernel(state, program, step_emb, lib_W, lib_b):
    raise NotImplementedError("write your pallas kernel here")


# speedup vs baseline: 9.2491x; 9.2491x over previous
import jax
import jax.numpy as jnp
from jax.experimental import pallas as pl

_BATCH = 16384
_STATE_DIM = 128
_NUM_STEPS = 50
_NUM_PRIMS = 16
_TILE = 2048


def _exec_kernel(program_ref, step_emb_ref, lib_W_ref, lib_b_ref,
                 state_ref, out_ref):
    p = jax.nn.softmax(program_ref[:], axis=-1)
    w = jnp.dot(p, lib_W_ref[:], preferred_element_type=jnp.float32)
    b = jnp.dot(p, lib_b_ref[:], preferred_element_type=jnp.float32)
    c = step_emb_ref[:] * w + b
    x = state_ref[:]
    for i in range(_NUM_STEPS):
        x = jnp.tanh(x * w[i] + c[i])
    out_ref[:] = x


def kernel(state, program, step_emb, lib_W, lib_b):
    grid = (_BATCH // _TILE,)
    final = pl.pallas_call(
        _exec_kernel,
        grid=grid,
        in_specs=[
            pl.BlockSpec((_NUM_STEPS, _NUM_PRIMS), lambda i: (0, 0)),
            pl.BlockSpec((_NUM_STEPS, _STATE_DIM), lambda i: (0, 0)),
            pl.BlockSpec((_NUM_PRIMS, _STATE_DIM), lambda i: (0, 0)),
            pl.BlockSpec((_NUM_PRIMS, _STATE_DIM), lambda i: (0, 0)),
            pl.BlockSpec((_TILE, _STATE_DIM), lambda i: (i, 0)),
        ],
        out_specs=pl.BlockSpec((_TILE, _STATE_DIM), lambda i: (i, 0)),
        out_shape=jax.ShapeDtypeStruct((_BATCH, _STATE_DIM), jnp.float32),
    )(program, step_emb, lib_W, lib_b, state)
    return (final, program)
